# Initial kernel scaffold; baseline (speedup 1.0000x reference)
import jax
import jax.numpy as jnp
from jax.experimental import pallas as pl


def kernel(in_feat, edge_index, w1s, w1n, w2s, w2n, w3s, w3n):
    raise NotImplementedError("write your pallas kernel here")


# speedup vs baseline: 6.0927x; 6.0927x over previous
import functools

import jax
import jax.numpy as jnp
from jax import lax
from jax.experimental import pallas as pl
from jax.experimental.pallas import tpu as pltpu
from jax.experimental.pallas import tpu_sc as plsc

_N = 10000
_NPAD = 10240
_E = 160000
_EB = 200


def _seg_sum_sc(x_t, src, dst, f_tile):
    info = plsc.get_sparse_core_info()
    nc, ns = info.num_cores, info.num_subcores
    nw = nc * ns
    e_per_w = _E // nw
    n_steps = e_per_w // _EB
    rows_per_s = _NPAD // ns

    mesh = plsc.VectorSubcoreMesh(core_axis_name="c", subcore_axis_name="s")

    @functools.partial(
        pl.kernel,
        mesh=mesh,
        out_type=jax.ShapeDtypeStruct((nc, _NPAD, f_tile), jnp.float32),
        scratch_types=[
            pltpu.VMEM((_EB,), jnp.int32),
            pltpu.VMEM((_EB,), jnp.int32),
            pltpu.VMEM((_EB, f_tile), jnp.float32),
            pltpu.VMEM_SHARED((_NPAD, f_tile), jnp.float32),
            pltpu.SemaphoreType.DMA,
        ],
    )
    def k(x_hbm, src_hbm, dst_hbm, z_hbm, out_hbm, src_v, dst_v, rows_v, acc_sh, sem):
        c = lax.axis_index("c")
        s = lax.axis_index("s")
        wid = s * nc + c
        pltpu.sync_copy(z_hbm.at[pl.ds(s * rows_per_s, rows_per_s)],
                        acc_sh.at[pl.ds(s * rows_per_s, rows_per_s)])
        plsc.subcore_barrier()

        def body(i, _):
            base = wid * e_per_w + i * _EB
            pltpu.sync_copy(src_hbm.at[pl.ds(base, _EB)], src_v)
            pltpu.sync_copy(dst_hbm.at[pl.ds(base, _EB)], dst_v)
            pltpu.async_copy(x_hbm.at[src_v], rows_v, sem).wait()
            pltpu.sync_copy(rows_v, acc_sh.at[dst_v], add=True)
            return _

        lax.fori_loop(0, n_steps, body, None)
        plsc.subcore_barrier()
        pltpu.sync_copy(acc_sh.at[pl.ds(s * rows_per_s, rows_per_s)],
                        out_hbm.at[c, pl.ds(s * rows_per_s, rows_per_s)])

    zeros = jnp.zeros((_NPAD, f_tile), jnp.float32)
    return k(x_t, src, dst, zeros)


def _seg_mean_and_deg(x2d, src, dst, need_deg, inv_deg):
    f = x2d.shape[1]
    f_pad = ((f + 127) // 128) * 128
    if f_pad != f:
        x2d = jnp.pad(x2d, ((0, 0), (0, f_pad - f)))
    tiles = []
    for t in range(f_pad // 128):
        part = _seg_sum_sc(x2d[:, t * 128:(t + 1) * 128], src, dst, 128)
        tiles.append(part[0, :_N] + part[1, :_N])
    summed = jnp.concatenate(tiles, axis=1)[:, :f]
    if need_deg:
        ones = jnp.ones((_N, 128), jnp.float32)
        dpart = _seg_sum_sc(ones, src, dst, 128)
        deg = dpart[0, :_N, 0] + dpart[1, :_N, 0]
        inv_deg = 1.0 / jnp.clip(deg, 1.0, None)
    return summed * inv_deg[:, None], inv_deg


def _mm_pool_kernel(s0, s1, s2, s3, m0, m1, m2, m3, ws, wn, o_ref):
    def term(sg, mg):
        a = jnp.dot(sg[...], ws[...], preferred_element_type=jnp.float32)
        a = a + jnp.dot(mg[...], wn[...], preferred_element_type=jnp.float32)
        return jnp.maximum(a, 0.0)

    r = jnp.maximum(jnp.maximum(term(s0, m0), term(s1, m1)),
                    jnp.maximum(term(s2, m2), term(s3, m3)))
    o_ref[...] = r


def _patches4(x, hp, wp):
    n, c = x.shape[0], x.shape[1]
    k = c * 9
    pt = lax.conv_general_dilated_patches(
        x, (3, 3), (1, 1), 'VALID',
        dimension_numbers=('NCHW', 'OIHW', 'NCHW'))
    pt = pt.transpose(0, 2, 3, 1)
    pt = pt.reshape(n, hp, 2, wp, 2, k)
    pt = pt.transpose(2, 4, 0, 1, 3, 5)
    return pt.reshape(4, n * hp * wp, k)


def _conv_relu_pool(x, mean, w_s, w_n, hp, wp, bm):
    n, cin = x.shape[0], x.shape[1]
    cout, k = w_s.shape[0], cin * 9
    ps = _patches4(x, hp, wp)
    pm = _patches4(mean, hp, wp)
    wsk = w_s.transpose(1, 2, 3, 0).reshape(k, cout)
    wnk = w_n.transpose(1, 2, 3, 0).reshape(k, cout)
    m4 = n * hp * wp
    grid = m4 // bm
    pspec = pl.BlockSpec((bm, k), lambda i: (i, 0))
    wspec = pl.BlockSpec((k, cout), lambda i: (0, 0))
    out = pl.pallas_call(
        _mm_pool_kernel,
        grid=(grid,),
        in_specs=[pspec] * 8 + [wspec] * 2,
        out_specs=pl.BlockSpec((bm, cout), lambda i: (i, 0)),
        out_shape=jax.ShapeDtypeStruct((m4, cout), jnp.float32),
    )(ps[0], ps[1], ps[2], ps[3], pm[0], pm[1], pm[2], pm[3], wsk, wnk)
    return out.reshape(n, hp, wp, cout).transpose(0, 3, 1, 2)


def kernel(in_feat, edge_index, w1s, w1n, w2s, w2n, w3s, w3n):
    src = edge_index[0].astype(jnp.int32)
    dst = edge_index[1].astype(jnp.int32)
    n = in_feat.shape[0]

    x = in_feat
    mean, inv_deg = _seg_mean_and_deg(x.reshape(n, -1), src, dst, True, None)
    h = _conv_relu_pool(x, mean.reshape(x.shape), w1s, w1n, 10, 10, 4000)

    mean, _ = _seg_mean_and_deg(h.reshape(n, -1), src, dst, False, inv_deg)
    h = _conv_relu_pool(h, mean.reshape(h.shape), w2s, w2n, 4, 4, 1000)

    mean, _ = _seg_mean_and_deg(h.reshape(n, -1), src, dst, False, inv_deg)
    h = _conv_relu_pool(h, mean.reshape(h.shape), w3s, w3n, 1, 1, 1000)

    return h.reshape(n, -1)
